# baseline (speedup 1.0000x reference)
import functools

import jax
import jax.numpy as jnp
from jax import lax
from jax.experimental import pallas as pl
from jax.experimental.pallas import tpu as pltpu
from jax.experimental.pallas import tpu_sc as plsc

N = 10000
E = 160000
D = 256
PAD = 10240
NROW = 80
NW = 32
EPW = 5008
E_PADDED = NW * EPW
NVREG = EPW // 16
NCHUNK = 40
DUMMY = PAD * PAD - 1
MM_ROWS = 256
NB = PAD // MM_ROWS


def _rank_body(pr_ref, out_ref):
    i = pl.program_id(0)
    j_f = pr_ref[...]
    a_f = pr_ref[pl.ds(i, 1), :]
    jb = lax.bitcast_convert_type(j_f, jnp.int32)
    ab = lax.bitcast_convert_type(a_f, jnp.int32)
    jb3 = jb[:, :, None]
    ab3 = ab[None, :, :]
    jj = (lax.broadcasted_iota(jnp.int32, (NROW, 128, 128), 0) * 128
          + lax.broadcasted_iota(jnp.int32, (NROW, 128, 128), 1))
    aa = i * 128 + lax.broadcasted_iota(jnp.int32, (NROW, 128, 128), 2)
    tie = jnp.where(jj < aa, 1, 0)
    keep = jnp.where(jb3 < ab3, 1, jnp.where(jb3 == ab3, tie, 0))
    cnt = jnp.sum(keep, axis=(0, 1))
    out_ref[0, 0, :] = cnt


@functools.cache
def _get_rank_call():
    return pl.pallas_call(
        _rank_body,
        grid=(NROW,),
        in_specs=[pl.BlockSpec((NROW, 128), lambda i: (0, 0))],
        out_specs=pl.BlockSpec((1, 1, 128), lambda i: (i, 0, 0)),
        out_shape=jax.ShapeDtypeStruct((NROW, 1, 128), jnp.int32),
    )


def _scatter_body(g0_hbm, g1_hbm, rank_hbm, s_hbm,
                  g0v, g1v, rankv, idxv, onesv, sem):
    wid = lax.axis_index("s") * 2 + lax.axis_index("c")
    base = wid * EPW
    pltpu.sync_copy(rank_hbm, rankv)
    pltpu.sync_copy(g0_hbm.at[pl.ds(base, EPW)], g0v)
    pltpu.sync_copy(g1_hbm.at[pl.ds(base, EPW)], g1v)

    for t in range(8):
        onesv[pl.ds(t * 16, 16)] = jnp.ones((16,), jnp.float32)
        idxv[NCHUNK - 1, pl.ds(t * 16, 16)] = jnp.full((16,), DUMMY, jnp.int32)

    @pl.loop(0, NVREG)
    def _(i):
        g0 = g0v[pl.ds(i * 16, 16)]
        g1 = g1v[pl.ds(i * 16, 16)]
        r0 = plsc.load_gather(rankv, [g0])
        r1 = plsc.load_gather(rankv, [g1])
        lo = jnp.minimum(r0, r1)
        hi = jnp.maximum(r0, r1)
        flat = lo * PAD + hi
        idxv[i // 8, pl.ds((i % 8) * 16, 16)] = flat

    @pl.loop(0, NCHUNK, step=8)
    def _(j0):
        descs = [
            pltpu.async_copy(onesv, s_hbm.at[idxv.at[j0 + t]], sem)
            for t in range(8)
        ]
        for d in descs:
            d.wait()


@functools.cache
def _get_scatter_call():
    return pl.kernel(
        _scatter_body,
        out_type=(),
        mesh=plsc.VectorSubcoreMesh(core_axis_name="c", subcore_axis_name="s"),
        compiler_params=pltpu.CompilerParams(needs_layout_passes=False),
        scratch_types=[
            pltpu.VMEM((EPW,), jnp.int32),
            pltpu.VMEM((EPW,), jnp.int32),
            pltpu.VMEM((PAD,), jnp.int32),
            pltpu.VMEM((NCHUNK, 128), jnp.int32),
            pltpu.VMEM((128,), jnp.float32),
            pltpu.SemaphoreType.DMA,
        ],
    )


def _mm_body(s_ref, ids_ref, am_ref, out_ref, acc_ref):
    i = pl.program_id(0)

    @pl.when(i == 0)
    def _():
        acc_ref[...] = jnp.zeros_like(acc_ref)

    sb = s_ref[0].astype(jnp.bfloat16)
    q = jnp.dot(sb, ids_ref[...],
                preferred_element_type=jnp.float32)
    ids_blk = ids_ref[pl.ds(i * MM_ROWS, MM_ROWS), :].astype(jnp.float32)
    acc_ref[0, :] += jnp.sum(ids_blk * q, axis=0)

    @pl.when(i == NB - 1)
    def _():
        enc = acc_ref[0, :]
        out_ref[0, :] = jnp.sum(am_ref[...] * enc[None, :], axis=1)


@functools.cache
def _get_mm_call():
    return pl.pallas_call(
        _mm_body,
        grid=(NB,),
        in_specs=[
            pl.BlockSpec((1, MM_ROWS, PAD), lambda i: (i, 0, 0)),
            pl.BlockSpec((PAD, D), lambda i: (0, 0)),
            pl.BlockSpec((10, D), lambda i: (0, 0)),
        ],
        out_specs=pl.BlockSpec((1, 10), lambda i: (0, 0)),
        out_shape=jax.ShapeDtypeStruct((1, 10), jnp.float32),
        scratch_shapes=[pltpu.VMEM((1, D), jnp.float32)],
    )


def kernel(x, edge_index, pr, ids_weight, am_weight):
    n = x.shape[0]
    pr_pad = jnp.concatenate(
        [pr, jnp.full((PAD - n,), jnp.inf, jnp.float32)]
    ).reshape(NROW, 128)
    rank = _get_rank_call()(pr_pad).reshape(PAD)

    epad = jnp.full((E_PADDED - E,), n, jnp.int32)
    g0 = jnp.concatenate([edge_index[0], epad])
    g1 = jnp.concatenate([edge_index[1], epad])

    s_ref = jax.new_ref(jnp.zeros((PAD * PAD,), jnp.float32))
    _get_scatter_call()(g0, g1, rank, s_ref)
    s3 = s_ref[...].reshape(NB, MM_ROWS, PAD)

    ids_pad = jnp.zeros((PAD, D), jnp.bfloat16).at[:n].set(
        ids_weight[:n].astype(jnp.bfloat16))
    return _get_mm_call()(s3, ids_pad, am_weight)


# speedup vs baseline: 1.0666x; 1.0666x over previous
import functools

import jax
import jax.numpy as jnp
from jax import lax
from jax.experimental import pallas as pl
from jax.experimental.pallas import tpu as pltpu
from jax.experimental.pallas import tpu_sc as plsc

N = 10000
E = 160000
D = 256
PAD = 10240
NROW = 80
NW = 32
EPW = 5008
E_PADDED = NW * EPW
NVREG = EPW // 16
NCHUNK = 40
DUMMY = PAD * PAD - 1
MM_ROWS = 256
NB = PAD // MM_ROWS


def _rank_body(pr_ref, out_ref, s0_ref):
    i = pl.program_id(0)
    s0_ref[...] = jnp.zeros((1, PAD, 128), jnp.float32)
    j_f = pr_ref[...]
    a_f = pr_ref[pl.ds(i, 1), :]
    jb = lax.bitcast_convert_type(j_f, jnp.int32)
    ab = lax.bitcast_convert_type(a_f, jnp.int32)
    jb3 = jb[:, :, None]
    ab3 = ab[None, :, :]
    jj = (lax.broadcasted_iota(jnp.int32, (NROW, 128, 128), 0) * 128
          + lax.broadcasted_iota(jnp.int32, (NROW, 128, 128), 1))
    aa = i * 128 + lax.broadcasted_iota(jnp.int32, (NROW, 128, 128), 2)
    tie = jnp.where(jj < aa, 1, 0)
    keep = jnp.where(jb3 < ab3, 1, jnp.where(jb3 == ab3, tie, 0))
    cnt = jnp.sum(keep, axis=(0, 1))
    out_ref[0, 0, :] = cnt


@functools.cache
def _get_rank_call():
    return pl.pallas_call(
        _rank_body,
        grid=(NROW,),
        in_specs=[pl.BlockSpec((NROW, 128), lambda i: (0, 0))],
        out_specs=[
            pl.BlockSpec((1, 1, 128), lambda i: (i, 0, 0)),
            pl.BlockSpec((1, PAD, 128), lambda i: (i, 0, 0)),
        ],
        out_shape=[
            jax.ShapeDtypeStruct((NROW, 1, 128), jnp.int32),
            jax.ShapeDtypeStruct((NROW, PAD, 128), jnp.float32),
        ],
    )


def _scatter_body(g0_hbm, g1_hbm, rank_hbm, s_hbm,
                  g0v, g1v, rankv, idxv, onesv, sem):
    wid = lax.axis_index("s") * 2 + lax.axis_index("c")
    base = wid * EPW
    pltpu.sync_copy(rank_hbm, rankv)
    pltpu.sync_copy(g0_hbm.at[pl.ds(base, EPW)], g0v)
    pltpu.sync_copy(g1_hbm.at[pl.ds(base, EPW)], g1v)

    for t in range(8):
        onesv[pl.ds(t * 16, 16)] = jnp.ones((16,), jnp.float32)
        idxv[NCHUNK - 1, pl.ds(t * 16, 16)] = jnp.full((16,), DUMMY, jnp.int32)

    @pl.loop(0, NVREG, unroll=4)
    def _(i):
        g0 = g0v[pl.ds(i * 16, 16)]
        g1 = g1v[pl.ds(i * 16, 16)]
        r0 = plsc.load_gather(rankv, [g0])
        r1 = plsc.load_gather(rankv, [g1])
        lo = jnp.minimum(r0, r1)
        hi = jnp.maximum(r0, r1)
        flat = lo * PAD + hi
        idxv[i // 8, pl.ds((i % 8) * 16, 16)] = flat

    @pl.loop(0, NCHUNK)
    def _(j):
        pltpu.async_copy(onesv, s_hbm.at[idxv.at[j]], sem)

    @pl.loop(0, NCHUNK)
    def _(j):
        pltpu.make_async_copy(onesv, s_hbm.at[idxv.at[0]], sem).wait()


@functools.cache
def _get_scatter_call():
    return pl.kernel(
        _scatter_body,
        out_type=(),
        mesh=plsc.VectorSubcoreMesh(core_axis_name="c", subcore_axis_name="s"),
        compiler_params=pltpu.CompilerParams(needs_layout_passes=False),
        scratch_types=[
            pltpu.VMEM((EPW,), jnp.int32),
            pltpu.VMEM((EPW,), jnp.int32),
            pltpu.VMEM((PAD,), jnp.int32),
            pltpu.VMEM((NCHUNK, 128), jnp.int32),
            pltpu.VMEM((128,), jnp.float32),
            pltpu.SemaphoreType.DMA,
        ],
    )


def _mm_body(s_ref, ids_ref, am_ref, out_ref, acc_ref):
    i = pl.program_id(0)

    @pl.when(i == 0)
    def _():
        acc_ref[...] = jnp.zeros_like(acc_ref)

    sb = s_ref[0].astype(jnp.bfloat16)
    q = jnp.dot(sb, ids_ref[...],
                preferred_element_type=jnp.float32)
    ids_blk = ids_ref[pl.ds(i * MM_ROWS, MM_ROWS), :].astype(jnp.float32)
    acc_ref[0, :] += jnp.sum(ids_blk * q, axis=0)

    @pl.when(i == NB - 1)
    def _():
        enc = acc_ref[0, :]
        out_ref[0, :] = jnp.sum(am_ref[...] * enc[None, :], axis=1)


@functools.cache
def _get_mm_call():
    return pl.pallas_call(
        _mm_body,
        grid=(NB,),
        in_specs=[
            pl.BlockSpec((1, MM_ROWS, PAD), lambda i: (i, 0, 0)),
            pl.BlockSpec((PAD, D), lambda i: (0, 0)),
            pl.BlockSpec((10, D), lambda i: (0, 0)),
        ],
        out_specs=pl.BlockSpec((1, 10), lambda i: (0, 0)),
        out_shape=jax.ShapeDtypeStruct((1, 10), jnp.float32),
        scratch_shapes=[pltpu.VMEM((1, D), jnp.float32)],
    )


def kernel(x, edge_index, pr, ids_weight, am_weight):
    n = x.shape[0]
    pr_pad = jnp.concatenate(
        [pr, jnp.full((PAD - n,), jnp.inf, jnp.float32)]
    ).reshape(NROW, 128)
    rank3, s0 = _get_rank_call()(pr_pad)
    rank = rank3.reshape(PAD)

    epad = jnp.full((E_PADDED - E,), n, jnp.int32)
    g0 = jnp.concatenate([edge_index[0], epad])
    g1 = jnp.concatenate([edge_index[1], epad])

    s_ref = jax.new_ref(s0.reshape(PAD * PAD))
    _get_scatter_call()(g0, g1, rank, s_ref)
    s3 = s_ref[...].reshape(NB, MM_ROWS, PAD)

    ids_pad = jnp.zeros((PAD, D), jnp.bfloat16).at[:n].set(
        ids_weight[:n].astype(jnp.bfloat16))
    return _get_mm_call()(s3, ids_pad, am_weight)
